# Initial kernel scaffold; baseline (speedup 1.0000x reference)
import jax
import jax.numpy as jnp
from jax.experimental import pallas as pl


def kernel(x, edge_index, W1_rel, b1, W1_root, W2_rel, b2, W2_root):
    raise NotImplementedError("write your pallas kernel here")


# speedup vs baseline: 5.7013x; 5.7013x over previous
import functools

import jax
import jax.numpy as jnp
from jax import lax
from jax.experimental import pallas as pl
from jax.experimental.pallas import tpu as pltpu
from jax.experimental.pallas import tpu_sc as plsc

N, E, D, H = 10000, 320000, 128, 32
NW = 32
EW = E // NW
K = 80
ITERS = EW // K
RPS = N // 16
ZR = 125
BN = 1000


def _sc_segsum(table, src, dst, width):
    mesh = plsc.VectorSubcoreMesh(core_axis_name="c", subcore_axis_name="s")

    @functools.partial(
        pl.kernel,
        mesh=mesh,
        out_type=jax.ShapeDtypeStruct((2, N, width), jnp.float32),
        compiler_params=pltpu.CompilerParams(use_tc_tiling_on_sc=False),
        scratch_types=[
            pltpu.VMEM((K,), jnp.int32),
            pltpu.VMEM((K,), jnp.int32),
            pltpu.VMEM((K, width), jnp.float32),
            pltpu.VMEM((ZR, width), jnp.float32),
            pltpu.VMEM_SHARED((N, width), jnp.float32),
            pltpu.SemaphoreType.DMA,
        ],
    )
    def k(tab_hbm, src_hbm, dst_hbm, out_hbm, sidx, didx, rows, zbuf, acc, sem):
        cid = lax.axis_index("c")
        sid = lax.axis_index("s")
        wid = sid * 2 + cid

        z = jnp.zeros((16,), jnp.float32)

        def zrow(i, carry):
            for c in range(width // 16):
                zbuf[i, pl.ds(c * 16, 16)] = z
            return carry

        lax.fori_loop(0, ZR, zrow, 0)

        def zcopy_w(c, carry):
            pltpu.sync_copy(zbuf, acc.at[pl.ds(sid * RPS + c * ZR, ZR)])
            return carry

        lax.fori_loop(0, RPS // ZR, zcopy_w, 0)
        plsc.subcore_barrier()

        eb = wid * EW

        def body(j, carry):
            off = eb + j * K
            pltpu.sync_copy(src_hbm.at[pl.ds(off, K)], sidx)
            pltpu.sync_copy(dst_hbm.at[pl.ds(off, K)], didx)
            pltpu.async_copy(tab_hbm.at[sidx], rows, sem).wait()
            pltpu.sync_copy(rows, acc.at[didx], add=True)
            return carry

        lax.fori_loop(0, ITERS, body, 0)
        plsc.subcore_barrier()

        @pl.when(sid == 0)
        def _():
            pltpu.sync_copy(acc, out_hbm.at[cid])

    return k(table, src, dst)


def _layer1_body(p_ref, x_ref, wrel_ref, wroot_ref, b_ref, h_ref):
    agg = p_ref[0] + p_ref[1]
    h_ref[...] = jnp.maximum(
        jnp.dot(agg, wrel_ref[...], preferred_element_type=jnp.float32)
        + b_ref[...]
        + jnp.dot(x_ref[...], wroot_ref[...], preferred_element_type=jnp.float32),
        0.0,
    )


def _tc_layer1(parts, x, W1_rel, W1_root, b1):
    return pl.pallas_call(
        _layer1_body,
        grid=(N // BN,),
        in_specs=[
            pl.BlockSpec((2, BN, D), lambda i: (0, i, 0)),
            pl.BlockSpec((BN, D), lambda i: (i, 0)),
            pl.BlockSpec((D, H), lambda i: (0, 0)),
            pl.BlockSpec((D, H), lambda i: (0, 0)),
            pl.BlockSpec((1, H), lambda i: (0, 0)),
        ],
        out_specs=pl.BlockSpec((BN, H), lambda i: (i, 0)),
        out_shape=jax.ShapeDtypeStruct((N, H), jnp.float32),
    )(parts, x, W1_rel, W1_root, b1.reshape(1, H))


def _layer2_body(p_ref, h_ref, wrel_ref, wroot_ref, b_ref, out_ref):
    agg = p_ref[0] + p_ref[1]
    out_ref[...] = jnp.maximum(
        jnp.dot(agg, wrel_ref[...], preferred_element_type=jnp.float32)
        + b_ref[...]
        + jnp.dot(h_ref[...], wroot_ref[...], preferred_element_type=jnp.float32),
        0.0,
    )


def _tc_layer2(parts, h, W2_rel, W2_root, b2):
    return pl.pallas_call(
        _layer2_body,
        grid=(N // BN,),
        in_specs=[
            pl.BlockSpec((2, BN, H), lambda i: (0, i, 0)),
            pl.BlockSpec((BN, H), lambda i: (i, 0)),
            pl.BlockSpec((H, 1), lambda i: (0, 0)),
            pl.BlockSpec((H, 1), lambda i: (0, 0)),
            pl.BlockSpec((1, 1), lambda i: (0, 0)),
        ],
        out_specs=pl.BlockSpec((BN, 1), lambda i: (i, 0)),
        out_shape=jax.ShapeDtypeStruct((N, 1), jnp.float32),
    )(parts, h, W2_rel, W2_root, b2.reshape(1, 1))


def kernel(x, edge_index, W1_rel, b1, W1_root, W2_rel, b2, W2_root):
    src = edge_index[0]
    dst = edge_index[1]
    parts1 = _sc_segsum(x, src, dst, D)
    h = _tc_layer1(parts1, x, W1_rel, W1_root, b1)
    parts2 = _sc_segsum(h, src, dst, H)
    out = _tc_layer2(parts2, h, W2_rel, W2_root, b2)
    return out


# speedup vs baseline: 13.7480x; 2.4114x over previous
import functools

import jax
import jax.numpy as jnp
from jax import lax
from jax.experimental import pallas as pl
from jax.experimental.pallas import tpu as pltpu
from jax.experimental.pallas import tpu_sc as plsc

N, E, D, H = 10000, 320000, 128, 32
NW = 32
EW = E // NW
K = 80
ITERS = EW // K
RPS = N // 16
ZR = 25
BN = 1000


def _sc_segsum(table, src3, dst3, width):
    mesh = plsc.VectorSubcoreMesh(core_axis_name="c", subcore_axis_name="s")

    @functools.partial(
        pl.kernel,
        mesh=mesh,
        out_type=jax.ShapeDtypeStruct((2, N, width), jnp.float32),
        compiler_params=pltpu.CompilerParams(use_tc_tiling_on_sc=False),
        scratch_types=[
            pltpu.VMEM((ITERS, K), jnp.int32),
            pltpu.VMEM((ITERS, K), jnp.int32),
            pltpu.VMEM((K, width), jnp.float32),
            pltpu.VMEM((K, width), jnp.float32),
            pltpu.VMEM((ZR, width), jnp.float32),
            pltpu.VMEM_SHARED((N, width), jnp.float32),
            pltpu.SemaphoreType.DMA,
            pltpu.SemaphoreType.DMA,
        ],
    )
    def k(tab_hbm, src_hbm, dst_hbm, out_hbm, sidx, didx, rows_a, rows_b,
          zbuf, acc, sem_a, sem_b):
        cid = lax.axis_index("c")
        sid = lax.axis_index("s")
        wid = sid * 2 + cid

        pltpu.async_copy(src_hbm.at[wid], sidx, sem_a)
        pltpu.async_copy(dst_hbm.at[wid], didx, sem_b)

        z = jnp.zeros((16,), jnp.float32)

        def zrow(i, carry):
            for c in range(width // 16):
                zbuf[i, pl.ds(c * 16, 16)] = z
            return carry

        lax.fori_loop(0, ZR, zrow, 0)

        def zcopy_w(c, carry):
            pltpu.sync_copy(zbuf, acc.at[pl.ds(sid * RPS + c * ZR, ZR)])
            return carry

        lax.fori_loop(0, RPS // ZR, zcopy_w, 0)
        pltpu.make_async_copy(src_hbm.at[wid], sidx, sem_a).wait()
        pltpu.make_async_copy(dst_hbm.at[wid], didx, sem_b).wait()
        plsc.subcore_barrier()

        def gather(j, buf, sem):
            pltpu.async_copy(tab_hbm.at[sidx.at[j]], buf, sem)

        def drain(buf, sem):
            pltpu.make_async_copy(tab_hbm.at[sidx.at[0]], buf, sem).wait()

        def scat(j, buf):
            pltpu.sync_copy(buf, acc.at[didx.at[j]], add=True)

        gather(0, rows_a, sem_a)

        def body(p, carry):
            j = 2 * p
            gather(j + 1, rows_b, sem_b)
            drain(rows_a, sem_a)
            scat(j, rows_a)
            gather(j + 2, rows_a, sem_a)
            drain(rows_b, sem_b)
            scat(j + 1, rows_b)
            return carry

        lax.fori_loop(0, (ITERS - 1) // 2, body, 0)
        drain(rows_a, sem_a)
        scat(ITERS - 1, rows_a)
        plsc.subcore_barrier()

        pltpu.sync_copy(acc.at[pl.ds(sid * RPS, RPS)],
                        out_hbm.at[cid, pl.ds(sid * RPS, RPS)])

    return k(table, src3, dst3)


def _layer1_body(p_ref, x_ref, wrel_ref, wroot_ref, b_ref, h_ref):
    agg = p_ref[0] + p_ref[1]
    h_ref[...] = jnp.maximum(
        jnp.dot(agg, wrel_ref[...], preferred_element_type=jnp.float32)
        + b_ref[...]
        + jnp.dot(x_ref[...], wroot_ref[...], preferred_element_type=jnp.float32),
        0.0,
    )


def _tc_layer1(parts, x, W1_rel, W1_root, b1):
    return pl.pallas_call(
        _layer1_body,
        grid=(N // BN,),
        in_specs=[
            pl.BlockSpec((2, BN, D), lambda i: (0, i, 0)),
            pl.BlockSpec((BN, D), lambda i: (i, 0)),
            pl.BlockSpec((D, H), lambda i: (0, 0)),
            pl.BlockSpec((D, H), lambda i: (0, 0)),
            pl.BlockSpec((1, H), lambda i: (0, 0)),
        ],
        out_specs=pl.BlockSpec((BN, H), lambda i: (i, 0)),
        out_shape=jax.ShapeDtypeStruct((N, H), jnp.float32),
    )(parts, x, W1_rel, W1_root, b1.reshape(1, H))


def _layer2_body(p_ref, h_ref, wrel_ref, wroot_ref, b_ref, out_ref):
    agg = p_ref[0] + p_ref[1]
    out_ref[...] = jnp.maximum(
        jnp.dot(agg, wrel_ref[...], preferred_element_type=jnp.float32)
        + b_ref[...]
        + jnp.dot(h_ref[...], wroot_ref[...], preferred_element_type=jnp.float32),
        0.0,
    )


def _tc_layer2(parts, h, W2_rel, W2_root, b2):
    return pl.pallas_call(
        _layer2_body,
        grid=(N // BN,),
        in_specs=[
            pl.BlockSpec((2, BN, H), lambda i: (0, i, 0)),
            pl.BlockSpec((BN, H), lambda i: (i, 0)),
            pl.BlockSpec((H, 1), lambda i: (0, 0)),
            pl.BlockSpec((H, 1), lambda i: (0, 0)),
            pl.BlockSpec((1, 1), lambda i: (0, 0)),
        ],
        out_specs=pl.BlockSpec((BN, 1), lambda i: (i, 0)),
        out_shape=jax.ShapeDtypeStruct((N, 1), jnp.float32),
    )(parts, h, W2_rel, W2_root, b2.reshape(1, 1))


def kernel(x, edge_index, W1_rel, b1, W1_root, W2_rel, b2, W2_root):
    src3 = edge_index[0].reshape(NW, ITERS, K)
    dst3 = edge_index[1].reshape(NW, ITERS, K)
    parts1 = _sc_segsum(x, src3, dst3, D)
    h = _tc_layer1(parts1, x, W1_rel, W1_root, b1)
    parts2 = _sc_segsum(h, src3, dst3, H)
    out = _tc_layer2(parts2, h, W2_rel, W2_root, b2)
    return out


# speedup vs baseline: 15.2537x; 1.1095x over previous
import functools

import jax
import jax.numpy as jnp
from jax import lax
from jax.experimental import pallas as pl
from jax.experimental.pallas import tpu as pltpu
from jax.experimental.pallas import tpu_sc as plsc

N, E, D, H = 10000, 320000, 128, 32
NW = 32
EW = E // NW
K = 80
ITERS = EW // K
RPS = N // 16
ZR = 25
BN = 1000


def _sc_segsum(table, src3, dst3, width, nbuf):
    mesh = plsc.VectorSubcoreMesh(core_axis_name="c", subcore_axis_name="s")

    @functools.partial(
        pl.kernel,
        mesh=mesh,
        out_type=jax.ShapeDtypeStruct((2, N, width), jnp.float32),
        compiler_params=pltpu.CompilerParams(use_tc_tiling_on_sc=False),
        scratch_types=(
            [pltpu.VMEM((ITERS, K), jnp.int32),
             pltpu.VMEM((ITERS, K), jnp.int32)]
            + [pltpu.VMEM((K, width), jnp.float32) for _ in range(nbuf)]
            + [pltpu.VMEM((ZR, width), jnp.float32),
               pltpu.VMEM_SHARED((N, width), jnp.float32)]
            + [pltpu.SemaphoreType.DMA for _ in range(nbuf)]
        ),
    )
    def k(tab_hbm, src_hbm, dst_hbm, out_hbm, sidx, didx, *rest):
        bufs = rest[:nbuf]
        zbuf = rest[nbuf]
        acc = rest[nbuf + 1]
        sems = rest[nbuf + 2:]
        cid = lax.axis_index("c")
        sid = lax.axis_index("s")
        wid = sid * 2 + cid

        pltpu.async_copy(src_hbm.at[wid], sidx, sems[0])
        pltpu.async_copy(dst_hbm.at[wid], didx, sems[1])

        z = jnp.zeros((16,), jnp.float32)

        def zrow(i, carry):
            for c in range(width // 16):
                zbuf[i, pl.ds(c * 16, 16)] = z
            return carry

        lax.fori_loop(0, ZR, zrow, 0)

        def zcopy_w(c, carry):
            pltpu.sync_copy(zbuf, acc.at[pl.ds(sid * RPS + c * ZR, ZR)])
            return carry

        lax.fori_loop(0, RPS // ZR, zcopy_w, 0)
        pltpu.make_async_copy(src_hbm.at[wid], sidx, sems[0]).wait()
        pltpu.make_async_copy(dst_hbm.at[wid], didx, sems[1]).wait()
        plsc.subcore_barrier()

        def gather(j, b):
            pltpu.async_copy(tab_hbm.at[sidx.at[j]], bufs[b], sems[b])

        def drain(b):
            pltpu.make_async_copy(tab_hbm.at[sidx.at[0]], bufs[b], sems[b]).wait()

        def scat(j, b):
            pltpu.sync_copy(bufs[b], acc.at[didx.at[j]], add=True)

        for b in range(nbuf - 1):
            gather(b, b)

        def body(p, carry):
            for b in range(nbuf):
                j = nbuf * p + b
                nxt = j + nbuf - 1

                @pl.when(nxt < ITERS)
                def _():
                    gather(nxt, (b + nbuf - 1) % nbuf)

                drain(b)
                scat(j, b)
            return carry

        lax.fori_loop(0, (ITERS - 1) // nbuf, body, 0)
        drain((ITERS - 1) % nbuf)
        scat(ITERS - 1, (ITERS - 1) % nbuf)
        plsc.subcore_barrier()

        pltpu.sync_copy(acc.at[pl.ds(sid * RPS, RPS)],
                        out_hbm.at[cid, pl.ds(sid * RPS, RPS)])

    return k(table, src3, dst3)


def _layer1_body(p_ref, x_ref, wrel_ref, wroot_ref, b_ref, h_ref):
    agg = p_ref[0] + p_ref[1]
    h_ref[...] = jnp.maximum(
        jnp.dot(agg, wrel_ref[...], preferred_element_type=jnp.float32)
        + b_ref[...]
        + jnp.dot(x_ref[...], wroot_ref[...], preferred_element_type=jnp.float32),
        0.0,
    )


def _tc_layer1(parts, x, W1_rel, W1_root, b1):
    return pl.pallas_call(
        _layer1_body,
        grid=(N // BN,),
        in_specs=[
            pl.BlockSpec((2, BN, D), lambda i: (0, i, 0)),
            pl.BlockSpec((BN, D), lambda i: (i, 0)),
            pl.BlockSpec((D, H), lambda i: (0, 0)),
            pl.BlockSpec((D, H), lambda i: (0, 0)),
            pl.BlockSpec((1, H), lambda i: (0, 0)),
        ],
        out_specs=pl.BlockSpec((BN, H), lambda i: (i, 0)),
        out_shape=jax.ShapeDtypeStruct((N, H), jnp.float32),
    )(parts, x, W1_rel, W1_root, b1.reshape(1, H))


def _layer2_body(p_ref, h_ref, wrel_ref, wroot_ref, b_ref, out_ref):
    agg = p_ref[0] + p_ref[1]
    out_ref[...] = jnp.maximum(
        jnp.dot(agg, wrel_ref[...], preferred_element_type=jnp.float32)
        + b_ref[...]
        + jnp.dot(h_ref[...], wroot_ref[...], preferred_element_type=jnp.float32),
        0.0,
    )


def _tc_layer2(parts, h, W2_rel, W2_root, b2):
    return pl.pallas_call(
        _layer2_body,
        grid=(N // BN,),
        in_specs=[
            pl.BlockSpec((2, BN, H), lambda i: (0, i, 0)),
            pl.BlockSpec((BN, H), lambda i: (i, 0)),
            pl.BlockSpec((H, 1), lambda i: (0, 0)),
            pl.BlockSpec((H, 1), lambda i: (0, 0)),
            pl.BlockSpec((1, 1), lambda i: (0, 0)),
        ],
        out_specs=pl.BlockSpec((BN, 1), lambda i: (i, 0)),
        out_shape=jax.ShapeDtypeStruct((N, 1), jnp.float32),
    )(parts, h, W2_rel, W2_root, b2.reshape(1, 1))


def kernel(x, edge_index, W1_rel, b1, W1_root, W2_rel, b2, W2_root):
    src3 = edge_index[0].reshape(NW, ITERS, K)
    dst3 = edge_index[1].reshape(NW, ITERS, K)
    parts1 = _sc_segsum(x, src3, dst3, D, nbuf=2)
    h = _tc_layer1(parts1, x, W1_rel, W1_root, b1)
    parts2 = _sc_segsum(h, src3, dst3, H, nbuf=4)
    out = _tc_layer2(parts2, h, W2_rel, W2_root, b2)
    return out


# speedup vs baseline: 17.0222x; 1.1159x over previous
import functools

import jax
import jax.numpy as jnp
from jax import lax
from jax.experimental import pallas as pl
from jax.experimental.pallas import tpu as pltpu
from jax.experimental.pallas import tpu_sc as plsc

N, E, D, H = 10000, 320000, 128, 32
NW = 32
EW = E // NW
K = 80
ITERS = EW // K
RPS = N // 16
ZR = 25
BN = 1000


def _sc_segsum(table, src3, dst3, width, nbuf):
    mesh = plsc.VectorSubcoreMesh(core_axis_name="c", subcore_axis_name="s")

    @functools.partial(
        pl.kernel,
        mesh=mesh,
        out_type=jax.ShapeDtypeStruct((2, N, width), jnp.float32),
        compiler_params=pltpu.CompilerParams(use_tc_tiling_on_sc=False),
        scratch_types=(
            [pltpu.VMEM((ITERS, K), jnp.int32),
             pltpu.VMEM((ITERS, K), jnp.int32)]
            + [pltpu.VMEM((K, width), jnp.float32) for _ in range(nbuf)]
            + [pltpu.VMEM_SHARED((N, width), jnp.float32)]
            + [pltpu.SemaphoreType.DMA for _ in range(nbuf)]
        ),
    )
    def k(tab_hbm, src_hbm, dst_hbm, out_hbm, sidx, didx, *rest):
        bufs = rest[:nbuf]
        acc = rest[nbuf]
        sems = rest[nbuf + 1:]
        cid = lax.axis_index("c")
        sid = lax.axis_index("s")
        wid = sid * 2 + cid

        pltpu.async_copy(src_hbm.at[wid], sidx, sems[0])
        pltpu.async_copy(dst_hbm.at[wid], didx, sems[1])

        z = jnp.zeros((16,), jnp.float32)

        def zrow(i, carry):
            for c in range(width // 16):
                bufs[0][i, pl.ds(c * 16, 16)] = z
            return carry

        lax.fori_loop(0, K, zrow, 0)

        for c in range(RPS // K):
            pltpu.sync_copy(bufs[0], acc.at[pl.ds(sid * RPS + c * K, K)])
        rem = RPS % K
        if rem:
            pltpu.sync_copy(bufs[0].at[pl.ds(0, rem)],
                            acc.at[pl.ds(sid * RPS + (RPS // K) * K, rem)])
        pltpu.make_async_copy(src_hbm.at[wid], sidx, sems[0]).wait()
        pltpu.make_async_copy(dst_hbm.at[wid], didx, sems[1]).wait()
        plsc.subcore_barrier()

        def gather(j, b):
            pltpu.async_copy(tab_hbm.at[sidx.at[j]], bufs[b], sems[b])

        def drain(b):
            pltpu.make_async_copy(tab_hbm.at[sidx.at[0]], bufs[b], sems[b]).wait()

        def scat(j, b):
            pltpu.sync_copy(bufs[b], acc.at[didx.at[j]], add=True)

        for b in range(nbuf - 1):
            gather(b, b)

        main = (ITERS - 1) // nbuf

        def body(p, carry):
            for b in range(nbuf):
                j = nbuf * p + b
                nxt = j + nbuf - 1

                @pl.when(nxt < ITERS)
                def _():
                    gather(nxt, (b + nbuf - 1) % nbuf)

                drain(b)
                scat(j, b)
            return carry

        lax.fori_loop(0, main, body, 0)
        for t in range(nbuf * main, ITERS):
            drain(t % nbuf)
            scat(t, t % nbuf)
        plsc.subcore_barrier()

        pltpu.sync_copy(acc.at[pl.ds(sid * RPS, RPS)],
                        out_hbm.at[cid, pl.ds(sid * RPS, RPS)])

    return k(table, src3, dst3)


def _layer1_body(p_ref, x_ref, wrel_ref, wroot_ref, b_ref, h_ref):
    agg = p_ref[0] + p_ref[1]
    h_ref[...] = jnp.maximum(
        jnp.dot(agg, wrel_ref[...], preferred_element_type=jnp.float32)
        + b_ref[...]
        + jnp.dot(x_ref[...], wroot_ref[...], preferred_element_type=jnp.float32),
        0.0,
    )


def _tc_layer1(parts, x, W1_rel, W1_root, b1):
    return pl.pallas_call(
        _layer1_body,
        grid=(N // BN,),
        in_specs=[
            pl.BlockSpec((2, BN, D), lambda i: (0, i, 0)),
            pl.BlockSpec((BN, D), lambda i: (i, 0)),
            pl.BlockSpec((D, H), lambda i: (0, 0)),
            pl.BlockSpec((D, H), lambda i: (0, 0)),
            pl.BlockSpec((1, H), lambda i: (0, 0)),
        ],
        out_specs=pl.BlockSpec((BN, H), lambda i: (i, 0)),
        out_shape=jax.ShapeDtypeStruct((N, H), jnp.float32),
    )(parts, x, W1_rel, W1_root, b1.reshape(1, H))


def _layer2_body(p_ref, h_ref, wrel_ref, wroot_ref, b_ref, out_ref):
    agg = p_ref[0] + p_ref[1]
    out_ref[...] = jnp.maximum(
        jnp.dot(agg, wrel_ref[...], preferred_element_type=jnp.float32)
        + b_ref[...]
        + jnp.dot(h_ref[...], wroot_ref[...], preferred_element_type=jnp.float32),
        0.0,
    )


def _tc_layer2(parts, h, W2_rel, W2_root, b2):
    return pl.pallas_call(
        _layer2_body,
        grid=(N // BN,),
        in_specs=[
            pl.BlockSpec((2, BN, H), lambda i: (0, i, 0)),
            pl.BlockSpec((BN, H), lambda i: (i, 0)),
            pl.BlockSpec((H, 1), lambda i: (0, 0)),
            pl.BlockSpec((H, 1), lambda i: (0, 0)),
            pl.BlockSpec((1, 1), lambda i: (0, 0)),
        ],
        out_specs=pl.BlockSpec((BN, 1), lambda i: (i, 0)),
        out_shape=jax.ShapeDtypeStruct((N, 1), jnp.float32),
    )(parts, h, W2_rel, W2_root, b2.reshape(1, 1))


def kernel(x, edge_index, W1_rel, b1, W1_root, W2_rel, b2, W2_root):
    src3 = edge_index[0].reshape(NW, ITERS, K)
    dst3 = edge_index[1].reshape(NW, ITERS, K)
    parts1 = _sc_segsum(x, src3, dst3, D, nbuf=3)
    h = _tc_layer1(parts1, x, W1_rel, W1_root, b1)
    parts2 = _sc_segsum(h, src3, dst3, H, nbuf=4)
    out = _tc_layer2(parts2, h, W2_rel, W2_root, b2)
    return out


# speedup vs baseline: 17.7533x; 1.0429x over previous
import functools

import jax
import jax.numpy as jnp
from jax import lax
from jax.experimental import pallas as pl
from jax.experimental.pallas import tpu as pltpu
from jax.experimental.pallas import tpu_sc as plsc

N, E, D, H = 10000, 320000, 128, 32
NW = 32
EW = E // NW
K = 80
ITERS = EW // K
RPS = N // 16
ZR = 25
BN = 1000


def _sc_segsum(table, src3, dst3, width, nbuf):
    mesh = plsc.VectorSubcoreMesh(core_axis_name="c", subcore_axis_name="s")

    @functools.partial(
        pl.kernel,
        mesh=mesh,
        out_type=jax.ShapeDtypeStruct((2, N, width), jnp.float32),
        compiler_params=pltpu.CompilerParams(use_tc_tiling_on_sc=False),
        scratch_types=(
            [pltpu.VMEM((ITERS, K), jnp.int32),
             pltpu.VMEM((ITERS, K), jnp.int32)]
            + [pltpu.VMEM((K, width), jnp.float32) for _ in range(nbuf)]
            + [pltpu.VMEM_SHARED((N, width), jnp.float32)]
            + [pltpu.SemaphoreType.DMA for _ in range(nbuf)]
        ),
    )
    def k(tab_hbm, src_hbm, dst_hbm, out_hbm, sidx, didx, *rest):
        bufs = rest[:nbuf]
        acc = rest[nbuf]
        sems = rest[nbuf + 1:]
        cid = lax.axis_index("c")
        sid = lax.axis_index("s")
        wid = sid * 2 + cid

        pltpu.async_copy(src_hbm.at[wid], sidx, sems[0])
        pltpu.async_copy(dst_hbm.at[wid], didx, sems[1])

        z = jnp.zeros((16,), jnp.float32)

        def zrow(i, carry):
            for c in range(width // 16):
                bufs[0][i, pl.ds(c * 16, 16)] = z
            return carry

        lax.fori_loop(0, K, zrow, 0)

        for c in range(RPS // K):
            pltpu.sync_copy(bufs[0], acc.at[pl.ds(sid * RPS + c * K, K)])
        rem = RPS % K
        if rem:
            pltpu.sync_copy(bufs[0].at[pl.ds(0, rem)],
                            acc.at[pl.ds(sid * RPS + (RPS // K) * K, rem)])
        pltpu.make_async_copy(src_hbm.at[wid], sidx, sems[0]).wait()
        pltpu.make_async_copy(dst_hbm.at[wid], didx, sems[1]).wait()
        plsc.subcore_barrier()

        def gather(j, b):
            pltpu.async_copy(tab_hbm.at[sidx.at[j]], bufs[b], sems[b])

        def drain(b):
            pltpu.make_async_copy(tab_hbm.at[sidx.at[0]], bufs[b], sems[b]).wait()

        def scat(j, b):
            pltpu.sync_copy(bufs[b], acc.at[didx.at[j]], add=True)

        for b in range(nbuf - 1):
            gather(b, b)

        main = (ITERS - 1) // nbuf

        def body(p, carry):
            for b in range(nbuf):
                j = nbuf * p + b
                nxt = j + nbuf - 1

                @pl.when(nxt < ITERS)
                def _():
                    gather(nxt, (b + nbuf - 1) % nbuf)

                drain(b)
                scat(j, b)
            return carry

        lax.fori_loop(0, main, body, 0)
        for t in range(nbuf * main, ITERS):
            drain(t % nbuf)
            scat(t, t % nbuf)
        plsc.subcore_barrier()

        pltpu.sync_copy(acc.at[pl.ds(sid * RPS, RPS)],
                        out_hbm.at[cid, pl.ds(sid * RPS, RPS)])

    return k(table, src3, dst3)


def _layer1_body(p_ref, x_ref, wrel_ref, wroot_ref, b_ref, h_ref):
    agg = p_ref[0] + p_ref[1]
    h_ref[...] = jnp.maximum(
        jnp.dot(agg, wrel_ref[...], preferred_element_type=jnp.float32)
        + b_ref[...]
        + jnp.dot(x_ref[...], wroot_ref[...], preferred_element_type=jnp.float32),
        0.0,
    )


def _tc_layer1(parts, x, W1_rel, W1_root, b1):
    return pl.pallas_call(
        _layer1_body,
        grid=(N // BN,),
        in_specs=[
            pl.BlockSpec((2, BN, D), lambda i: (0, i, 0)),
            pl.BlockSpec((BN, D), lambda i: (i, 0)),
            pl.BlockSpec((D, H), lambda i: (0, 0)),
            pl.BlockSpec((D, H), lambda i: (0, 0)),
            pl.BlockSpec((1, H), lambda i: (0, 0)),
        ],
        out_specs=pl.BlockSpec((BN, H), lambda i: (i, 0)),
        out_shape=jax.ShapeDtypeStruct((N, H), jnp.float32),
    )(parts, x, W1_rel, W1_root, b1.reshape(1, H))


def _layer2_body(p_ref, h_ref, wrel_ref, wroot_ref, b_ref, out_ref):
    agg = p_ref[0] + p_ref[1]
    out_ref[...] = jnp.maximum(
        jnp.dot(agg, wrel_ref[...], preferred_element_type=jnp.float32)
        + b_ref[...]
        + jnp.dot(h_ref[...], wroot_ref[...], preferred_element_type=jnp.float32),
        0.0,
    )


def _tc_layer2(parts, h, W2_rel, W2_root, b2):
    return pl.pallas_call(
        _layer2_body,
        grid=(N // BN,),
        in_specs=[
            pl.BlockSpec((2, BN, H), lambda i: (0, i, 0)),
            pl.BlockSpec((BN, H), lambda i: (i, 0)),
            pl.BlockSpec((H, 1), lambda i: (0, 0)),
            pl.BlockSpec((H, 1), lambda i: (0, 0)),
            pl.BlockSpec((1, 1), lambda i: (0, 0)),
        ],
        out_specs=pl.BlockSpec((BN, 1), lambda i: (i, 0)),
        out_shape=jax.ShapeDtypeStruct((N, 1), jnp.float32),
    )(parts, h, W2_rel, W2_root, b2.reshape(1, 1))


def kernel(x, edge_index, W1_rel, b1, W1_root, W2_rel, b2, W2_root):
    src3 = edge_index[0].reshape(NW, ITERS, K)
    dst3 = edge_index[1].reshape(NW, ITERS, K)
    parts1 = _sc_segsum(x, src3, dst3, D, nbuf=3)
    h = _tc_layer1(parts1, x, W1_rel, W1_root, b1)
    parts2 = _sc_segsum(h, src3, dst3, H, nbuf=8)
    out = _tc_layer2(parts2, h, W2_rel, W2_root, b2)
    return out
